# baseline (speedup 1.0000x reference)
import functools

import jax
import jax.numpy as jnp
from jax import lax
from jax.experimental import pallas as pl
from jax.experimental.pallas import tpu as pltpu
from jax.experimental.pallas import tpu_sc as plsc

VOCAB = 1000
VPAD = 1024
LPR = VPAD // 128
T = 20
TPAD = 24
NC = 2
NS = 16
NW = NC * NS


def _make_gather(b):
    ktok = T // 2
    kr = ktok * LPR
    b_per_w = b // NW
    nchunk = 2 * b_per_w
    assert nchunk % 2 == 0 and kr <= 128
    rows_per_b = TPAD * LPR
    mesh = plsc.VectorSubcoreMesh(core_axis_name="c", subcore_axis_name="s")

    @functools.partial(
        pl.kernel,
        out_type=jax.ShapeDtypeStruct((b * rows_per_b, 128), jnp.float32),
        mesh=mesh,
        scratch_types=[
            pltpu.VMEM((b_per_w * T * LPR,), jnp.int32),
            pltpu.VMEM((2, kr, 128), jnp.float32),
            pltpu.SemaphoreType.DMA,
            pltpu.SemaphoreType.DMA,
        ],
        compiler_params=pltpu.CompilerParams(use_tc_tiling_on_sc=False),
    )
    def gather_kernel(tview_hbm, idx_hbm, out_hbm, idx_v, rows_v, sem0, sem1):
        wid = lax.axis_index("s") * NC + lax.axis_index("c")
        idx_base = wid * b_per_w * T * LPR
        out_base = wid * b_per_w * rows_per_b
        sems = (sem0, sem1)
        pltpu.sync_copy(idx_hbm.at[pl.ds(idx_base, b_per_w * T * LPR)], idx_v)

        def gather_dma(c, slot):
            return pltpu.make_async_copy(
                tview_hbm.at[idx_v.at[pl.ds(c * kr, kr)]],
                rows_v.at[slot],
                sems[slot],
            )

        def out_copy(c, slot):
            off = out_base + (c // 2) * rows_per_b + (c % 2) * kr
            pltpu.sync_copy(rows_v.at[slot], out_hbm.at[pl.ds(off, kr)])

        gather_dma(0, 0).start()

        def body(c2, carry):
            c = 2 * c2
            gather_dma(c + 1, 1).start()
            gather_dma(c, 0).wait()
            out_copy(c, 0)
            gather_dma(c + 2, 0).start()
            gather_dma(c + 1, 1).wait()
            out_copy(c + 1, 1)
            return carry

        lax.fori_loop(0, nchunk // 2 - 1, body, 0)
        c = nchunk - 2
        gather_dma(c + 1, 1).start()
        gather_dma(c, 0).wait()
        out_copy(c, 0)
        gather_dma(c + 1, 1).wait()
        out_copy(c + 1, 1)

    return gather_kernel


def _relayout_body(x_ref, o_ref):
    o_ref[...] = x_ref[:, :T, :VOCAB]


def _make_relayout(b, bb):
    return pl.pallas_call(
        _relayout_body,
        grid=(b // bb,),
        in_specs=[pl.BlockSpec((bb, TPAD, VPAD), lambda i: (i, 0, 0))],
        out_specs=pl.BlockSpec((bb, T, VOCAB), lambda i: (i, 0, 0)),
        out_shape=jax.ShapeDtypeStruct((b, T, VOCAB), jnp.float32),
    )


_gather = _make_gather(4096)
_relayout = _make_relayout(4096, 32)


@jax.jit
def kernel(idx, token_embedding_table):
    b, t = idx.shape
    flat = idx.reshape(b * t)
    idx8 = (flat[:, None] * LPR + jnp.arange(LPR, dtype=jnp.int32)).reshape(-1)
    table_p = jnp.pad(token_embedding_table, ((0, 0), (0, VPAD - VOCAB)))
    tview = table_p.reshape(VOCAB * LPR, 128)
    mid = _gather(tview, idx8)
    return _relayout(mid.reshape(b, TPAD, VPAD))
